# Initial kernel scaffold; baseline (speedup 1.0000x reference)
import jax
import jax.numpy as jnp
from jax.experimental import pallas as pl


def kernel(x, edge_index, W1, b1, g1, be1, W2, b2, g2, be2, LW1, Lb1, LW2, Lb2):
    raise NotImplementedError("write your pallas kernel here")


# speedup vs baseline: 8.2136x; 8.2136x over previous
import functools

import jax
import jax.numpy as jnp
from jax import lax
from jax.experimental import pallas as pl
from jax.experimental.pallas import tpu as pltpu
from jax.experimental.pallas import tpu_sc as plsc

N = 10000
NP = 10240
D = 128
E = 320000
NCORE = 2
NSUB = 16
NW = NCORE * NSUB
CH = 128
NCH = 80
NSTRIP = 2
CPS = NCH // NSTRIP
EP = NW * NCH * CH
RPT = NP // NSUB
RB = 512
GRID = NP // RB


def _mesh():
    return plsc.VectorSubcoreMesh(core_axis_name="c", subcore_axis_name="s")


def _deg_body(dst_hbm, ones_hbm, zrow_hbm, out_hbm, idx_v, ones_v, acc_sh):
    c = lax.axis_index("c")
    s = lax.axis_index("s")
    w = c * NSUB + s
    pltpu.sync_copy(zrow_hbm, acc_sh.at[pl.ds(s * RPT, RPT)])
    pltpu.sync_copy(ones_hbm, ones_v)
    pltpu.sync_copy(dst_hbm.at[pl.ds(w * NCH, NCH)], idx_v)
    plsc.subcore_barrier()

    @pl.loop(0, NCH)
    def _(j):
        pltpu.sync_copy(ones_v, acc_sh.at[idx_v.at[j]], add=True)

    plsc.subcore_barrier()
    pltpu.sync_copy(acc_sh.at[pl.ds(s * RPT, RPT)],
                    out_hbm.at[pl.ds(c * NP + s * RPT, RPT)])


_deg_call = functools.partial(
    pl.kernel,
    out_type=jax.ShapeDtypeStruct((NCORE * NP,), jnp.float32),
    mesh=_mesh(),
    scratch_types=[
        pltpu.VMEM((NCH, CH), jnp.int32),
        pltpu.VMEM((CH,), jnp.float32),
        pltpu.VMEM_SHARED((NP,), jnp.float32),
    ],
)(_deg_body)


def _spmm_body(hp_hbm, src_hbm, dst_hbm, zrows_hbm, out_hbm,
               srci_v, dsti_v, rows_v, acc_sh, sem):
    c = lax.axis_index("c")
    s = lax.axis_index("s")
    w = c * NSUB + s
    pltpu.sync_copy(zrows_hbm, acc_sh.at[pl.ds(s * RPT, RPT)])
    plsc.subcore_barrier()

    for strip in range(NSTRIP):
        base = w * NCH + strip * CPS
        pltpu.sync_copy(src_hbm.at[pl.ds(base, CPS)], srci_v)
        pltpu.sync_copy(dst_hbm.at[pl.ds(base, CPS)], dsti_v)
        pltpu.make_async_copy(hp_hbm.at[srci_v.at[0]], rows_v.at[0],
                              sem).start()

        @pl.loop(0, CPS // 2)
        def _(jj):
            for b in range(2):
                j = jj * 2 + b
                pltpu.make_async_copy(hp_hbm.at[srci_v.at[j]], rows_v.at[b],
                                      sem).wait()

                @pl.when(j + 1 < CPS)
                def _():
                    pltpu.make_async_copy(hp_hbm.at[srci_v.at[j + 1]],
                                          rows_v.at[1 - b], sem).start()

                pltpu.sync_copy(rows_v.at[b], acc_sh.at[dsti_v.at[j]],
                                add=True)

    plsc.subcore_barrier()
    pltpu.sync_copy(acc_sh.at[pl.ds(s * RPT, RPT)],
                    out_hbm.at[c, pl.ds(s * RPT, RPT), :])


_spmm_call = functools.partial(
    pl.kernel,
    out_type=jax.ShapeDtypeStruct((NCORE, NP, D), jnp.float32),
    mesh=_mesh(),
    scratch_types=[
        pltpu.VMEM((CPS, CH), jnp.int32),
        pltpu.VMEM((CPS, CH), jnp.int32),
        pltpu.VMEM((2, CH, D), jnp.float32),
        pltpu.VMEM_SHARED((NP, D), jnp.float32),
        pltpu.SemaphoreType.DMA,
    ],
)(_spmm_body)


def _tc_a_body(x_ref, w_ref, da_ref, db_ref, hp_ref, dinv_ref):
    di = lax.rsqrt(da_ref[...] + db_ref[...] + 1.0)
    dinv_ref[...] = di
    t = jnp.dot(x_ref[...], w_ref[...], preferred_element_type=jnp.float32)
    hp_ref[...] = t * di


def _tc_c_body(p0_ref, p1_ref, hp1_ref, di_ref, w2_ref, s1_ref, c1_ref,
               x1_ref, hp2_ref):
    di = di_ref[...]
    tot = (p0_ref[...] + p1_ref[...] + hp1_ref[...]) * di
    x1 = jnp.maximum(tot * s1_ref[...] + c1_ref[...], 0.0)
    x1_ref[...] = x1
    hp2_ref[...] = jnp.dot(x1, w2_ref[...],
                           preferred_element_type=jnp.float32) * di


def _tc_e_body(q0_ref, q1_ref, hp2_ref, di_ref, x1_ref, s2_ref, c2_ref,
               la_ref, lb_ref, lb1_ref, lw2_ref, lb2_ref, out_ref):
    di = di_ref[...]
    x2 = jnp.maximum(((q0_ref[...] + q1_ref[...] + hp2_ref[...]) * di)
                     * s2_ref[...] + c2_ref[...], 0.0)
    h3 = jnp.dot(x1_ref[...], la_ref[...], preferred_element_type=jnp.float32)
    h3 = h3 + jnp.dot(x2, lb_ref[...], preferred_element_type=jnp.float32)
    h3 = jnp.maximum(h3 + lb1_ref[...], 0.0)
    out_ref[...] = (jnp.dot(h3, lw2_ref[...],
                            preferred_element_type=jnp.float32) + lb2_ref[...])


def _row_spec():
    return pl.BlockSpec((RB, D), lambda i: (i, 0))


def _col_spec():
    return pl.BlockSpec((RB, 1), lambda i: (i, 0))


def _mat_spec():
    return pl.BlockSpec((D, D), lambda i: (0, 0))


def _vec_spec():
    return pl.BlockSpec((1, D), lambda i: (0, 0))


def kernel(x, edge_index, W1, b1, g1, be1, W2, b2, g2, be2, LW1, Lb1, LW2, Lb2):
    f32 = jnp.float32
    pad_e = jnp.full((EP - E,), NP - 1, jnp.int32)
    src2d = jnp.concatenate([edge_index[0], pad_e]).reshape(EP // CH, CH)
    dst2d = jnp.concatenate([edge_index[1], pad_e]).reshape(EP // CH, CH)
    x_pad = jnp.pad(x, ((0, NP - N), (0, 0)))
    ones_r = jnp.ones((CH,), f32)
    zrow = jnp.zeros((RPT,), f32)
    zrows = jnp.zeros((RPT, D), f32)

    rs = jax.lax.rsqrt(f32(1.0 + 1e-5))
    s1v = g1 * rs
    c1v = b1 * s1v + be1
    s2v = g2 * rs
    c2v = b2 * s2v + be2
    s1 = s1v[None, :]
    c1 = c1v[None, :]
    s2 = s2v[None, :]
    c2 = c2v[None, :]
    la = LW1[:D]
    lb = LW1[D:]
    lb1 = Lb1[None, :]
    lb2 = Lb2[None, :]

    degs = _deg_call(dst2d, ones_r, zrow)
    dega = degs[:NP, None]
    degb = degs[NP:, None]

    hp1, dinv = pl.pallas_call(
        _tc_a_body,
        grid=(GRID,),
        in_specs=[_row_spec(), _mat_spec(), _col_spec(), _col_spec()],
        out_specs=[_row_spec(), _col_spec()],
        out_shape=[jax.ShapeDtypeStruct((NP, D), f32),
                   jax.ShapeDtypeStruct((NP, 1), f32)],
    )(x_pad, W1, dega, degb)

    p = _spmm_call(hp1, src2d, dst2d, zrows)

    x1, hp2 = pl.pallas_call(
        _tc_c_body,
        grid=(GRID,),
        in_specs=[_row_spec(), _row_spec(), _row_spec(), _col_spec(),
                  _mat_spec(), _vec_spec(), _vec_spec()],
        out_specs=[_row_spec(), _row_spec()],
        out_shape=[jax.ShapeDtypeStruct((NP, D), f32),
                   jax.ShapeDtypeStruct((NP, D), f32)],
    )(p[0], p[1], hp1, dinv, W2, s1, c1)

    q = _spmm_call(hp2, src2d, dst2d, zrows)

    out = pl.pallas_call(
        _tc_e_body,
        grid=(GRID,),
        in_specs=[_row_spec(), _row_spec(), _row_spec(), _col_spec(),
                  _row_spec(), _vec_spec(), _vec_spec(),
                  _mat_spec(), _mat_spec(), _vec_spec(),
                  _mat_spec(), _vec_spec()],
        out_specs=_row_spec(),
        out_shape=jax.ShapeDtypeStruct((NP, D), f32),
    )(q[0], q[1], hp2, dinv, x1, s2, c2, la, lb, lb1, LW2, lb2)

    return out[:N]


# speedup vs baseline: 24.9508x; 3.0377x over previous
import functools

import jax
import jax.numpy as jnp
from jax import lax
from jax.experimental import pallas as pl
from jax.experimental.pallas import tpu as pltpu
from jax.experimental.pallas import tpu_sc as plsc

N = 10000
NP = 10240
D = 128
E = 320000
NCORE = 2
NSUB = 16
NW = NCORE * NSUB
CH = 128
NCH = 80
NSTRIP = 2
CPS = NCH // NSTRIP
EP = NW * NCH * CH
RPT = NP // NSUB
RB = 512
GRID = NP // RB


def _mesh():
    return plsc.VectorSubcoreMesh(core_axis_name="c", subcore_axis_name="s")


def _deg_body(dst_hbm, ones_hbm, zrow_hbm, out_hbm, idx_v, ones_v, acc_sh):
    c = lax.axis_index("c")
    s = lax.axis_index("s")
    w = c * NSUB + s
    pltpu.sync_copy(zrow_hbm, acc_sh.at[pl.ds(s * RPT, RPT)])
    pltpu.sync_copy(ones_hbm, ones_v)
    pltpu.sync_copy(dst_hbm.at[pl.ds(w * NCH, NCH)], idx_v)
    plsc.subcore_barrier()

    @pl.loop(0, NCH)
    def _(j):
        pltpu.sync_copy(ones_v, acc_sh.at[idx_v.at[j]], add=True)

    plsc.subcore_barrier()
    pltpu.sync_copy(acc_sh.at[pl.ds(s * RPT, RPT)],
                    out_hbm.at[pl.ds(c * NP + s * RPT, RPT)])


_deg_call = functools.partial(
    pl.kernel,
    out_type=jax.ShapeDtypeStruct((NCORE * NP,), jnp.float32),
    mesh=_mesh(),
    scratch_types=[
        pltpu.VMEM((NCH, CH), jnp.int32),
        pltpu.VMEM((CH,), jnp.float32),
        pltpu.VMEM_SHARED((NP,), jnp.float32),
    ],
)(_deg_body)


def _spmm_body(hp_hbm, src_hbm, dst_hbm, zrows_hbm, out_hbm,
               srci_v, dsti_v, rows_v, acc_sh, sem):
    c = lax.axis_index("c")
    s = lax.axis_index("s")
    w = c * NSUB + s
    pltpu.sync_copy(zrows_hbm, acc_sh.at[pl.ds(s * RPT, RPT)])
    plsc.subcore_barrier()

    for strip in range(NSTRIP):
        base = w * NCH + strip * CPS
        pltpu.sync_copy(src_hbm.at[pl.ds(base, CPS)], srci_v)
        pltpu.sync_copy(dst_hbm.at[pl.ds(base, CPS)], dsti_v)
        pltpu.make_async_copy(hp_hbm.at[srci_v.at[0]], rows_v.at[0],
                              sem).start()

        @pl.loop(0, CPS // 2)
        def _(jj):
            for b in range(2):
                j = jj * 2 + b
                pltpu.make_async_copy(hp_hbm.at[srci_v.at[j]], rows_v.at[b],
                                      sem).wait()

                @pl.when(j + 1 < CPS)
                def _():
                    pltpu.make_async_copy(hp_hbm.at[srci_v.at[j + 1]],
                                          rows_v.at[1 - b], sem).start()

                pltpu.sync_copy(rows_v.at[b], acc_sh.at[dsti_v.at[j]],
                                add=True)

    plsc.subcore_barrier()
    pltpu.sync_copy(acc_sh.at[pl.ds(s * RPT, RPT)],
                    out_hbm.at[c, pl.ds(s * RPT, RPT), :])


_spmm_call = functools.partial(
    pl.kernel,
    out_type=jax.ShapeDtypeStruct((NCORE, NP, D), jnp.float32),
    mesh=_mesh(),
    scratch_types=[
        pltpu.VMEM((CPS, CH), jnp.int32),
        pltpu.VMEM((CPS, CH), jnp.int32),
        pltpu.VMEM((2, CH, D), jnp.float32),
        pltpu.VMEM_SHARED((NP, D), jnp.float32),
        pltpu.SemaphoreType.DMA,
    ],
)(_spmm_body)


def _tc_a_body(x_ref, w_ref, da_ref, db_ref, hp_ref, dinv_ref):
    di = lax.rsqrt(da_ref[...] + db_ref[...] + 1.0)
    dinv_ref[...] = di
    t = jnp.dot(x_ref[...], w_ref[...], preferred_element_type=jnp.float32)
    hp_ref[...] = t * di


def _tc_c_body(p0_ref, p1_ref, hp1_ref, di_ref, w2_ref, s1_ref, c1_ref,
               x1_ref, hp2_ref):
    di = di_ref[...]
    tot = (p0_ref[...] + p1_ref[...] + hp1_ref[...]) * di
    x1 = jnp.maximum(tot * s1_ref[...] + c1_ref[...], 0.0)
    x1_ref[...] = x1
    hp2_ref[...] = jnp.dot(x1, w2_ref[...],
                           preferred_element_type=jnp.float32) * di


def _tc_e_body(q0_ref, q1_ref, hp2_ref, di_ref, x1_ref, s2_ref, c2_ref,
               la_ref, lb_ref, lb1_ref, lw2_ref, lb2_ref, out_ref):
    di = di_ref[...]
    x2 = jnp.maximum(((q0_ref[...] + q1_ref[...] + hp2_ref[...]) * di)
                     * s2_ref[...] + c2_ref[...], 0.0)
    h3 = jnp.dot(x1_ref[...], la_ref[...], preferred_element_type=jnp.float32)
    h3 = h3 + jnp.dot(x2, lb_ref[...], preferred_element_type=jnp.float32)
    h3 = jnp.maximum(h3 + lb1_ref[...], 0.0)
    out_ref[...] = (jnp.dot(h3, lw2_ref[...],
                            preferred_element_type=jnp.float32) + lb2_ref[...])


def _row_spec():
    return pl.BlockSpec((RB, D), lambda i: (i, 0))


def _col_spec():
    return pl.BlockSpec((RB, 1), lambda i: (i, 0))


def _mat_spec():
    return pl.BlockSpec((D, D), lambda i: (0, 0))


def _vec_spec():
    return pl.BlockSpec((1, D), lambda i: (0, 0))


def kernel(x, edge_index, W1, b1, g1, be1, W2, b2, g2, be2, LW1, Lb1, LW2, Lb2):
    f32 = jnp.float32
    pad_e = N + (jnp.arange(EP - E, dtype=jnp.int32) % (NP - N))
    src2d = jnp.concatenate([edge_index[0], pad_e]).reshape(EP // CH, CH)
    dst2d = jnp.concatenate([edge_index[1], pad_e]).reshape(EP // CH, CH)
    x_pad = jnp.pad(x, ((0, NP - N), (0, 0)))
    ones_r = jnp.ones((CH,), f32)
    zrow = jnp.zeros((RPT,), f32)
    zrows = jnp.zeros((RPT, D), f32)

    rs = jax.lax.rsqrt(f32(1.0 + 1e-5))
    s1v = g1 * rs
    c1v = b1 * s1v + be1
    s2v = g2 * rs
    c2v = b2 * s2v + be2
    s1 = s1v[None, :]
    c1 = c1v[None, :]
    s2 = s2v[None, :]
    c2 = c2v[None, :]
    la = LW1[:D]
    lb = LW1[D:]
    lb1 = Lb1[None, :]
    lb2 = Lb2[None, :]

    degs = _deg_call(dst2d, ones_r, zrow)
    dega = degs[:NP, None]
    degb = degs[NP:, None]

    hp1, dinv = pl.pallas_call(
        _tc_a_body,
        grid=(GRID,),
        in_specs=[_row_spec(), _mat_spec(), _col_spec(), _col_spec()],
        out_specs=[_row_spec(), _col_spec()],
        out_shape=[jax.ShapeDtypeStruct((NP, D), f32),
                   jax.ShapeDtypeStruct((NP, 1), f32)],
    )(x_pad, W1, dega, degb)

    p = _spmm_call(hp1, src2d, dst2d, zrows)

    x1, hp2 = pl.pallas_call(
        _tc_c_body,
        grid=(GRID,),
        in_specs=[_row_spec(), _row_spec(), _row_spec(), _col_spec(),
                  _mat_spec(), _vec_spec(), _vec_spec()],
        out_specs=[_row_spec(), _row_spec()],
        out_shape=[jax.ShapeDtypeStruct((NP, D), f32),
                   jax.ShapeDtypeStruct((NP, D), f32)],
    )(p[0], p[1], hp1, dinv, W2, s1, c1)

    q = _spmm_call(hp2, src2d, dst2d, zrows)

    out = pl.pallas_call(
        _tc_e_body,
        grid=(GRID,),
        in_specs=[_row_spec(), _row_spec(), _row_spec(), _col_spec(),
                  _row_spec(), _vec_spec(), _vec_spec(),
                  _mat_spec(), _mat_spec(), _vec_spec(),
                  _mat_spec(), _vec_spec()],
        out_specs=_row_spec(),
        out_shape=jax.ShapeDtypeStruct((NP, D), f32),
    )(q[0], q[1], hp2, dinv, x1, s2, c2, la, lb, lb1, LW2, lb2)

    return out[:N]


# speedup vs baseline: 28.0771x; 1.1253x over previous
import functools

import jax
import jax.numpy as jnp
from jax import lax
from jax.experimental import pallas as pl
from jax.experimental.pallas import tpu as pltpu
from jax.experimental.pallas import tpu_sc as plsc

N = 10000
NP = 10240
D = 128
E = 320000
NCORE = 2
NSUB = 16
NW = NCORE * NSUB
CH = 128
NCH = 80
NSTRIP = 2
CPS = NCH // NSTRIP
EP = NW * NCH * CH
RPT = NP // NSUB
RB = 512
GRID = NP // RB


def _mesh():
    return plsc.VectorSubcoreMesh(core_axis_name="c", subcore_axis_name="s")


def _deg_body(dst_hbm, ones_hbm, zrow_hbm, out_hbm, idx_v, ones_v, acc_sh):
    c = lax.axis_index("c")
    s = lax.axis_index("s")
    w = c * NSUB + s
    pltpu.sync_copy(zrow_hbm, acc_sh.at[pl.ds(s * RPT, RPT)])
    pltpu.sync_copy(ones_hbm, ones_v)
    pltpu.sync_copy(dst_hbm.at[pl.ds(w * NCH, NCH)], idx_v)
    plsc.subcore_barrier()

    @pl.loop(0, NCH)
    def _(j):
        pltpu.sync_copy(ones_v, acc_sh.at[idx_v.at[j]], add=True)

    plsc.subcore_barrier()
    pltpu.sync_copy(acc_sh.at[pl.ds(s * RPT, RPT)],
                    out_hbm.at[pl.ds(c * NP + s * RPT, RPT)])


_deg_call = functools.partial(
    pl.kernel,
    out_type=jax.ShapeDtypeStruct((NCORE * NP,), jnp.float32),
    mesh=_mesh(),
    scratch_types=[
        pltpu.VMEM((NCH, CH), jnp.int32),
        pltpu.VMEM((CH,), jnp.float32),
        pltpu.VMEM_SHARED((NP,), jnp.float32),
    ],
)(_deg_body)


def _spmm_body(hp_hbm, src_hbm, dst_hbm, zrows_hbm, out_hbm,
               srci_v, dsti_v, rows_v, acc_sh, sem):
    c = lax.axis_index("c")
    s = lax.axis_index("s")
    w = c * NSUB + s
    pltpu.sync_copy(zrows_hbm, acc_sh.at[pl.ds(s * RPT, RPT)])
    plsc.subcore_barrier()

    for strip in range(NSTRIP):
        base = w * NCH + strip * CPS
        pltpu.sync_copy(src_hbm.at[pl.ds(base, CPS)], srci_v)
        pltpu.sync_copy(dst_hbm.at[pl.ds(base, CPS)], dsti_v)
        pltpu.make_async_copy(hp_hbm.at[srci_v.at[0]], rows_v.at[0],
                              sem).start()
        pltpu.make_async_copy(hp_hbm.at[srci_v.at[1]], rows_v.at[1],
                              sem).start()

        @pl.loop(0, CPS // 2)
        def _(jj):
            for b in range(2):
                j = jj * 2 + b
                pltpu.make_async_copy(hp_hbm.at[srci_v.at[j]], rows_v.at[b],
                                      sem).wait()
                pltpu.sync_copy(rows_v.at[b], acc_sh.at[dsti_v.at[j]],
                                add=True)

                @pl.when(j + 2 < CPS)
                def _():
                    pltpu.make_async_copy(hp_hbm.at[srci_v.at[j + 2]],
                                          rows_v.at[b], sem).start()

    plsc.subcore_barrier()
    pltpu.sync_copy(acc_sh.at[pl.ds(s * RPT, RPT)],
                    out_hbm.at[c, pl.ds(s * RPT, RPT), :])


_spmm_call = functools.partial(
    pl.kernel,
    out_type=jax.ShapeDtypeStruct((NCORE, NP, D), jnp.float32),
    mesh=_mesh(),
    scratch_types=[
        pltpu.VMEM((CPS, CH), jnp.int32),
        pltpu.VMEM((CPS, CH), jnp.int32),
        pltpu.VMEM((2, CH, D), jnp.float32),
        pltpu.VMEM_SHARED((NP, D), jnp.float32),
        pltpu.SemaphoreType.DMA,
    ],
)(_spmm_body)


def _tc_a_body(x_ref, w_ref, da_ref, db_ref, hp_ref, dinv_ref):
    di = lax.rsqrt(da_ref[...] + db_ref[...] + 1.0)
    dinv_ref[...] = di
    t = jnp.dot(x_ref[...], w_ref[...], preferred_element_type=jnp.float32)
    hp_ref[...] = t * di


def _tc_c_body(p0_ref, p1_ref, hp1_ref, di_ref, w2_ref, s1_ref, c1_ref,
               x1_ref, hp2_ref):
    di = di_ref[...]
    tot = (p0_ref[...] + p1_ref[...] + hp1_ref[...]) * di
    x1 = jnp.maximum(tot * s1_ref[...] + c1_ref[...], 0.0)
    x1_ref[...] = x1
    hp2_ref[...] = jnp.dot(x1, w2_ref[...],
                           preferred_element_type=jnp.float32) * di


def _tc_e_body(q0_ref, q1_ref, hp2_ref, di_ref, x1_ref, s2_ref, c2_ref,
               la_ref, lb_ref, lb1_ref, lw2_ref, lb2_ref, out_ref):
    di = di_ref[...]
    x2 = jnp.maximum(((q0_ref[...] + q1_ref[...] + hp2_ref[...]) * di)
                     * s2_ref[...] + c2_ref[...], 0.0)
    h3 = jnp.dot(x1_ref[...], la_ref[...], preferred_element_type=jnp.float32)
    h3 = h3 + jnp.dot(x2, lb_ref[...], preferred_element_type=jnp.float32)
    h3 = jnp.maximum(h3 + lb1_ref[...], 0.0)
    out_ref[...] = (jnp.dot(h3, lw2_ref[...],
                            preferred_element_type=jnp.float32) + lb2_ref[...])


def _row_spec():
    return pl.BlockSpec((RB, D), lambda i: (i, 0))


def _col_spec():
    return pl.BlockSpec((RB, 1), lambda i: (i, 0))


def _mat_spec():
    return pl.BlockSpec((D, D), lambda i: (0, 0))


def _vec_spec():
    return pl.BlockSpec((1, D), lambda i: (0, 0))


def kernel(x, edge_index, W1, b1, g1, be1, W2, b2, g2, be2, LW1, Lb1, LW2, Lb2):
    f32 = jnp.float32
    pad_e = N + (jnp.arange(EP - E, dtype=jnp.int32) % (NP - N))
    src2d = jnp.concatenate([edge_index[0], pad_e]).reshape(EP // CH, CH)
    dst2d = jnp.concatenate([edge_index[1], pad_e]).reshape(EP // CH, CH)
    x_pad = jnp.pad(x, ((0, NP - N), (0, 0)))
    ones_r = jnp.ones((CH,), f32)
    zrow = jnp.zeros((RPT,), f32)
    zrows = jnp.zeros((RPT, D), f32)

    rs = jax.lax.rsqrt(f32(1.0 + 1e-5))
    s1v = g1 * rs
    c1v = b1 * s1v + be1
    s2v = g2 * rs
    c2v = b2 * s2v + be2
    s1 = s1v[None, :]
    c1 = c1v[None, :]
    s2 = s2v[None, :]
    c2 = c2v[None, :]
    la = LW1[:D]
    lb = LW1[D:]
    lb1 = Lb1[None, :]
    lb2 = Lb2[None, :]

    degs = _deg_call(dst2d, ones_r, zrow)
    dega = degs[:NP, None]
    degb = degs[NP:, None]

    hp1, dinv = pl.pallas_call(
        _tc_a_body,
        grid=(GRID,),
        in_specs=[_row_spec(), _mat_spec(), _col_spec(), _col_spec()],
        out_specs=[_row_spec(), _col_spec()],
        out_shape=[jax.ShapeDtypeStruct((NP, D), f32),
                   jax.ShapeDtypeStruct((NP, 1), f32)],
    )(x_pad, W1, dega, degb)

    p = _spmm_call(hp1, src2d, dst2d, zrows)

    x1, hp2 = pl.pallas_call(
        _tc_c_body,
        grid=(GRID,),
        in_specs=[_row_spec(), _row_spec(), _row_spec(), _col_spec(),
                  _mat_spec(), _vec_spec(), _vec_spec()],
        out_specs=[_row_spec(), _row_spec()],
        out_shape=[jax.ShapeDtypeStruct((NP, D), f32),
                   jax.ShapeDtypeStruct((NP, D), f32)],
    )(p[0], p[1], hp1, dinv, W2, s1, c1)

    q = _spmm_call(hp2, src2d, dst2d, zrows)

    out = pl.pallas_call(
        _tc_e_body,
        grid=(GRID,),
        in_specs=[_row_spec(), _row_spec(), _row_spec(), _col_spec(),
                  _row_spec(), _vec_spec(), _vec_spec(),
                  _mat_spec(), _mat_spec(), _vec_spec(),
                  _mat_spec(), _vec_spec()],
        out_specs=_row_spec(),
        out_shape=jax.ShapeDtypeStruct((NP, D), f32),
    )(q[0], q[1], hp2, dinv, x1, s2, c2, la, lb, lb1, LW2, lb2)

    return out[:N]


# speedup vs baseline: 28.3217x; 1.0087x over previous
import functools

import jax
import jax.numpy as jnp
from jax import lax
from jax.experimental import pallas as pl
from jax.experimental.pallas import tpu as pltpu
from jax.experimental.pallas import tpu_sc as plsc

N = 10000
NP = 10240
D = 128
E = 320000
NCORE = 2
NSUB = 16
NW = NCORE * NSUB
CH = 128
NCH = 80
NSTRIP = 2
CPS = NCH // NSTRIP
EP = NW * NCH * CH
RPT = NP // NSUB
RB = 512
GRID = NP // RB


def _mesh():
    return plsc.VectorSubcoreMesh(core_axis_name="c", subcore_axis_name="s")


def _deg_body(dst_hbm, ones_hbm, zrow_hbm, out_hbm, idx_v, ones_v, acc_sh):
    c = lax.axis_index("c")
    s = lax.axis_index("s")
    w = c * NSUB + s
    pltpu.sync_copy(zrow_hbm, acc_sh.at[pl.ds(s * RPT, RPT)])
    pltpu.sync_copy(ones_hbm, ones_v)
    pltpu.sync_copy(dst_hbm.at[pl.ds(w * NCH, NCH)], idx_v)
    plsc.subcore_barrier()

    @pl.loop(0, NCH)
    def _(j):
        pltpu.sync_copy(ones_v, acc_sh.at[idx_v.at[j]], add=True)

    plsc.subcore_barrier()
    pltpu.sync_copy(acc_sh.at[pl.ds(s * RPT, RPT)],
                    out_hbm.at[pl.ds(c * NP + s * RPT, RPT)])


_deg_call = functools.partial(
    pl.kernel,
    out_type=jax.ShapeDtypeStruct((NCORE * NP,), jnp.float32),
    mesh=_mesh(),
    scratch_types=[
        pltpu.VMEM((NCH, CH), jnp.int32),
        pltpu.VMEM((CH,), jnp.float32),
        pltpu.VMEM_SHARED((NP,), jnp.float32),
    ],
)(_deg_body)


def _spmm_body(hp_hbm, src_hbm, dst_hbm, zrows_hbm, out_hbm,
               srci_v, dsti_v, rows_v, acc_sh, sem):
    c = lax.axis_index("c")
    s = lax.axis_index("s")
    w = c * NSUB + s

    @pl.when(c == 0)
    def _():
        pltpu.sync_copy(hp_hbm.at[pl.ds(s * RPT, RPT)],
                        acc_sh.at[pl.ds(s * RPT, RPT)])

    @pl.when(c == 1)
    def _():
        pltpu.sync_copy(zrows_hbm, acc_sh.at[pl.ds(s * RPT, RPT)])

    plsc.subcore_barrier()

    for strip in range(NSTRIP):
        base = w * NCH + strip * CPS
        pltpu.sync_copy(src_hbm.at[pl.ds(base, CPS)], srci_v)
        pltpu.sync_copy(dst_hbm.at[pl.ds(base, CPS)], dsti_v)
        pltpu.make_async_copy(hp_hbm.at[srci_v.at[0]], rows_v.at[0],
                              sem).start()
        pltpu.make_async_copy(hp_hbm.at[srci_v.at[1]], rows_v.at[1],
                              sem).start()

        @pl.loop(0, CPS // 2)
        def _(jj):
            for b in range(2):
                j = jj * 2 + b
                pltpu.make_async_copy(hp_hbm.at[srci_v.at[j]], rows_v.at[b],
                                      sem).wait()
                pltpu.sync_copy(rows_v.at[b], acc_sh.at[dsti_v.at[j]],
                                add=True)

                @pl.when(j + 2 < CPS)
                def _():
                    pltpu.make_async_copy(hp_hbm.at[srci_v.at[j + 2]],
                                          rows_v.at[b], sem).start()

    plsc.subcore_barrier()
    pltpu.sync_copy(acc_sh.at[pl.ds(s * RPT, RPT)],
                    out_hbm.at[c, pl.ds(s * RPT, RPT), :])


_spmm_call = functools.partial(
    pl.kernel,
    out_type=jax.ShapeDtypeStruct((NCORE, NP, D), jnp.float32),
    mesh=_mesh(),
    scratch_types=[
        pltpu.VMEM((CPS, CH), jnp.int32),
        pltpu.VMEM((CPS, CH), jnp.int32),
        pltpu.VMEM((2, CH, D), jnp.float32),
        pltpu.VMEM_SHARED((NP, D), jnp.float32),
        pltpu.SemaphoreType.DMA,
    ],
)(_spmm_body)


def _tc_a0_body(x_ref, w_ref, t_ref):
    t_ref[...] = jnp.dot(x_ref[...], w_ref[...],
                         preferred_element_type=jnp.float32)


def _tc_a1_body(t_ref, da_ref, db_ref, hp_ref, dinv_ref):
    di = lax.rsqrt(da_ref[...] + db_ref[...] + 1.0)
    dinv_ref[...] = di
    hp_ref[...] = t_ref[...] * di


def _tc_c_body(p0_ref, p1_ref, di_ref, w2_ref, s1_ref, c1_ref,
               x1_ref, hp2_ref):
    di = di_ref[...]
    tot = (p0_ref[...] + p1_ref[...]) * di
    x1 = jnp.maximum(tot * s1_ref[...] + c1_ref[...], 0.0)
    x1_ref[...] = x1
    hp2_ref[...] = jnp.dot(x1, w2_ref[...],
                           preferred_element_type=jnp.float32) * di


def _tc_e_body(q0_ref, q1_ref, di_ref, x1_ref, s2_ref, c2_ref,
               la_ref, lb_ref, lb1_ref, lw2_ref, lb2_ref, out_ref):
    di = di_ref[...]
    x2 = jnp.maximum(((q0_ref[...] + q1_ref[...]) * di)
                     * s2_ref[...] + c2_ref[...], 0.0)
    h3 = jnp.dot(x1_ref[...], la_ref[...], preferred_element_type=jnp.float32)
    h3 = h3 + jnp.dot(x2, lb_ref[...], preferred_element_type=jnp.float32)
    h3 = jnp.maximum(h3 + lb1_ref[...], 0.0)
    out_ref[...] = (jnp.dot(h3, lw2_ref[...],
                            preferred_element_type=jnp.float32) + lb2_ref[...])


def _row_spec():
    return pl.BlockSpec((RB, D), lambda i: (i, 0))


def _col_spec():
    return pl.BlockSpec((RB, 1), lambda i: (i, 0))


def _mat_spec():
    return pl.BlockSpec((D, D), lambda i: (0, 0))


def _vec_spec():
    return pl.BlockSpec((1, D), lambda i: (0, 0))


def kernel(x, edge_index, W1, b1, g1, be1, W2, b2, g2, be2, LW1, Lb1, LW2, Lb2):
    f32 = jnp.float32
    pad_e = N + (jnp.arange(EP - E, dtype=jnp.int32) % (NP - N))
    src2d = jnp.concatenate([edge_index[0], pad_e]).reshape(EP // CH, CH)
    dst2d = jnp.concatenate([edge_index[1], pad_e]).reshape(EP // CH, CH)
    x_pad = jnp.pad(x, ((0, NP - N), (0, 0)))
    ones_r = jnp.ones((CH,), f32)
    zrow = jnp.zeros((RPT,), f32)
    zrows = jnp.zeros((RPT, D), f32)

    rs = jax.lax.rsqrt(f32(1.0 + 1e-5))
    s1v = g1 * rs
    c1v = b1 * s1v + be1
    s2v = g2 * rs
    c2v = b2 * s2v + be2
    s1 = s1v[None, :]
    c1 = c1v[None, :]
    s2 = s2v[None, :]
    c2 = c2v[None, :]
    la = LW1[:D]
    lb = LW1[D:]
    lb1 = Lb1[None, :]
    lb2 = Lb2[None, :]

    degs = _deg_call(dst2d, ones_r, zrow)
    t1 = pl.pallas_call(
        _tc_a0_body,
        grid=(GRID,),
        in_specs=[_row_spec(), _mat_spec()],
        out_specs=_row_spec(),
        out_shape=jax.ShapeDtypeStruct((NP, D), f32),
    )(x_pad, W1)
    dega = degs[:NP, None]
    degb = degs[NP:, None]

    hp1, dinv = pl.pallas_call(
        _tc_a1_body,
        grid=(GRID,),
        in_specs=[_row_spec(), _col_spec(), _col_spec()],
        out_specs=[_row_spec(), _col_spec()],
        out_shape=[jax.ShapeDtypeStruct((NP, D), f32),
                   jax.ShapeDtypeStruct((NP, 1), f32)],
    )(t1, dega, degb)

    p = _spmm_call(hp1, src2d, dst2d, zrows)

    x1, hp2 = pl.pallas_call(
        _tc_c_body,
        grid=(GRID,),
        in_specs=[_row_spec(), _row_spec(), _col_spec(),
                  _mat_spec(), _vec_spec(), _vec_spec()],
        out_specs=[_row_spec(), _row_spec()],
        out_shape=[jax.ShapeDtypeStruct((NP, D), f32),
                   jax.ShapeDtypeStruct((NP, D), f32)],
    )(p[0], p[1], dinv, W2, s1, c1)

    q = _spmm_call(hp2, src2d, dst2d, zrows)

    out = pl.pallas_call(
        _tc_e_body,
        grid=(GRID,),
        in_specs=[_row_spec(), _row_spec(), _col_spec(),
                  _row_spec(), _vec_spec(), _vec_spec(),
                  _mat_spec(), _mat_spec(), _vec_spec(),
                  _mat_spec(), _vec_spec()],
        out_specs=_row_spec(),
        out_shape=jax.ShapeDtypeStruct((NP, D), f32),
    )(q[0], q[1], dinv, x1, s2, c2, la, lb, lb1, LW2, lb2)

    return out[:N]


# speedup vs baseline: 28.4097x; 1.0031x over previous
import functools

import jax
import jax.numpy as jnp
from jax import lax
from jax.experimental import pallas as pl
from jax.experimental.pallas import tpu as pltpu
from jax.experimental.pallas import tpu_sc as plsc

N = 10000
NP = 10240
D = 128
E = 320000
NCORE = 2
NSUB = 16
NW = NCORE * NSUB
CH = 128
NCH = 80
NSTRIP = 2
CPS = NCH // NSTRIP
EP = NW * NCH * CH
RPT = NP // NSUB
RB = 512
GRID = NP // RB


def _mesh():
    return plsc.VectorSubcoreMesh(core_axis_name="c", subcore_axis_name="s")


def _deg_body(dst_hbm, ones_hbm, zrow_hbm, out_hbm, idx_v, ones_v, acc_sh):
    c = lax.axis_index("c")
    s = lax.axis_index("s")
    w = c * NSUB + s
    pltpu.sync_copy(zrow_hbm, acc_sh.at[pl.ds(s * RPT, RPT)])
    pltpu.sync_copy(ones_hbm, ones_v)
    pltpu.sync_copy(dst_hbm.at[pl.ds(w * NCH, NCH)], idx_v)
    plsc.subcore_barrier()

    @pl.loop(0, NCH)
    def _(j):
        pltpu.sync_copy(ones_v, acc_sh.at[idx_v.at[j]], add=True)

    plsc.subcore_barrier()
    pltpu.sync_copy(acc_sh.at[pl.ds(s * RPT, RPT)],
                    out_hbm.at[pl.ds(c * NP + s * RPT, RPT)])


_deg_call = functools.partial(
    pl.kernel,
    out_type=jax.ShapeDtypeStruct((NCORE * NP,), jnp.float32),
    mesh=_mesh(),
    scratch_types=[
        pltpu.VMEM((NCH, CH), jnp.int32),
        pltpu.VMEM((CH,), jnp.float32),
        pltpu.VMEM_SHARED((NP,), jnp.float32),
    ],
)(_deg_body)


def _spmm_body(hp_hbm, src_hbm, dst_hbm, zrows_hbm, out_hbm,
               srci_v, dsti_v, rows_v, acc_sh, sem):
    c = lax.axis_index("c")
    s = lax.axis_index("s")
    w = c * NSUB + s

    @pl.when(c == 0)
    def _():
        pltpu.sync_copy(hp_hbm.at[pl.ds(s * RPT, RPT)],
                        acc_sh.at[pl.ds(s * RPT, RPT)])

    @pl.when(c == 1)
    def _():
        pltpu.sync_copy(zrows_hbm, acc_sh.at[pl.ds(s * RPT, RPT)])

    plsc.subcore_barrier()

    for strip in range(NSTRIP):
        base = w * NCH + strip * CPS
        pltpu.sync_copy(src_hbm.at[pl.ds(base, CPS)], srci_v)
        pltpu.sync_copy(dst_hbm.at[pl.ds(base, CPS)], dsti_v)
        pltpu.make_async_copy(hp_hbm.at[srci_v.at[0]], rows_v.at[0],
                              sem).start()
        pltpu.make_async_copy(hp_hbm.at[srci_v.at[1]], rows_v.at[1],
                              sem).start()

        @pl.loop(0, CPS // 2)
        def _(jj):
            for b in range(2):
                j = jj * 2 + b
                pltpu.make_async_copy(hp_hbm.at[srci_v.at[j]], rows_v.at[b],
                                      sem).wait()
                pltpu.sync_copy(rows_v.at[b], acc_sh.at[dsti_v.at[j]],
                                add=True)

                @pl.when(j + 2 < CPS)
                def _():
                    pltpu.make_async_copy(hp_hbm.at[srci_v.at[j + 2]],
                                          rows_v.at[b], sem).start()

    plsc.subcore_barrier()
    pltpu.sync_copy(acc_sh.at[pl.ds(s * RPT, RPT)],
                    out_hbm.at[c, pl.ds(s * RPT, RPT), :])


_spmm_call = functools.partial(
    pl.kernel,
    out_type=jax.ShapeDtypeStruct((NCORE, NP, D), jnp.float32),
    mesh=_mesh(),
    scratch_types=[
        pltpu.VMEM((CPS, CH), jnp.int32),
        pltpu.VMEM((CPS, CH), jnp.int32),
        pltpu.VMEM((2, CH, D), jnp.float32),
        pltpu.VMEM_SHARED((NP, D), jnp.float32),
        pltpu.SemaphoreType.DMA,
    ],
)(_spmm_body)


def _tc_a_body(x_ref, w_ref, da_ref, db_ref, hp_ref, dinv_ref):
    di = lax.rsqrt(da_ref[...] + db_ref[...] + 1.0)
    dinv_ref[...] = di
    t = jnp.dot(x_ref[...], w_ref[...], preferred_element_type=jnp.float32)
    hp_ref[...] = t * di


def _tc_c_body(p0_ref, p1_ref, di_ref, w2_ref, s1_ref, c1_ref,
               x1_ref, hp2_ref):
    di = di_ref[...]
    tot = (p0_ref[...] + p1_ref[...]) * di
    x1 = jnp.maximum(tot * s1_ref[...] + c1_ref[...], 0.0)
    x1_ref[...] = x1
    hp2_ref[...] = jnp.dot(x1, w2_ref[...],
                           preferred_element_type=jnp.float32) * di


def _tc_e_body(q0_ref, q1_ref, di_ref, x1_ref, s2_ref, c2_ref,
               la_ref, lb_ref, lb1_ref, lw2_ref, lb2_ref, out_ref):
    di = di_ref[...]
    x2 = jnp.maximum(((q0_ref[...] + q1_ref[...]) * di)
                     * s2_ref[...] + c2_ref[...], 0.0)
    h3 = jnp.dot(x1_ref[...], la_ref[...], preferred_element_type=jnp.float32)
    h3 = h3 + jnp.dot(x2, lb_ref[...], preferred_element_type=jnp.float32)
    h3 = jnp.maximum(h3 + lb1_ref[...], 0.0)
    out_ref[...] = (jnp.dot(h3, lw2_ref[...],
                            preferred_element_type=jnp.float32) + lb2_ref[...])


def _row_spec():
    return pl.BlockSpec((RB, D), lambda i: (i, 0))


def _col_spec():
    return pl.BlockSpec((RB, 1), lambda i: (i, 0))


def _mat_spec():
    return pl.BlockSpec((D, D), lambda i: (0, 0))


def _vec_spec():
    return pl.BlockSpec((1, D), lambda i: (0, 0))


def kernel(x, edge_index, W1, b1, g1, be1, W2, b2, g2, be2, LW1, Lb1, LW2, Lb2):
    f32 = jnp.float32
    pad_e = N + (jnp.arange(EP - E, dtype=jnp.int32) % (NP - N))
    src2d = jnp.concatenate([edge_index[0], pad_e]).reshape(EP // CH, CH)
    dst2d = jnp.concatenate([edge_index[1], pad_e]).reshape(EP // CH, CH)
    x_pad = jnp.pad(x, ((0, NP - N), (0, 0)))
    ones_r = jnp.ones((CH,), f32)
    zrow = jnp.zeros((RPT,), f32)
    zrows = jnp.zeros((RPT, D), f32)

    rs = jax.lax.rsqrt(f32(1.0 + 1e-5))
    s1v = g1 * rs
    c1v = b1 * s1v + be1
    s2v = g2 * rs
    c2v = b2 * s2v + be2
    s1 = s1v[None, :]
    c1 = c1v[None, :]
    s2 = s2v[None, :]
    c2 = c2v[None, :]
    la = LW1[:D]
    lb = LW1[D:]
    lb1 = Lb1[None, :]
    lb2 = Lb2[None, :]

    degs = _deg_call(dst2d, ones_r, zrow)
    dega = degs[:NP, None]
    degb = degs[NP:, None]

    hp1, dinv = pl.pallas_call(
        _tc_a_body,
        grid=(GRID,),
        in_specs=[_row_spec(), _mat_spec(), _col_spec(), _col_spec()],
        out_specs=[_row_spec(), _col_spec()],
        out_shape=[jax.ShapeDtypeStruct((NP, D), f32),
                   jax.ShapeDtypeStruct((NP, 1), f32)],
    )(x_pad, W1, dega, degb)

    p = _spmm_call(hp1, src2d, dst2d, zrows)

    x1, hp2 = pl.pallas_call(
        _tc_c_body,
        grid=(GRID,),
        in_specs=[_row_spec(), _row_spec(), _col_spec(),
                  _mat_spec(), _vec_spec(), _vec_spec()],
        out_specs=[_row_spec(), _row_spec()],
        out_shape=[jax.ShapeDtypeStruct((NP, D), f32),
                   jax.ShapeDtypeStruct((NP, D), f32)],
    )(p[0], p[1], dinv, W2, s1, c1)

    q = _spmm_call(hp2, src2d, dst2d, zrows)

    out = pl.pallas_call(
        _tc_e_body,
        grid=(GRID,),
        in_specs=[_row_spec(), _row_spec(), _col_spec(),
                  _row_spec(), _vec_spec(), _vec_spec(),
                  _mat_spec(), _mat_spec(), _vec_spec(),
                  _mat_spec(), _vec_spec()],
        out_specs=_row_spec(),
        out_shape=jax.ShapeDtypeStruct((NP, D), f32),
    )(q[0], q[1], dinv, x1, s2, c2, la, lb, lb1, LW2, lb2)

    return out[:N]
